# baseline (speedup 1.0000x reference)
import jax
import jax.numpy as jnp
from jax.experimental import pallas as pl
from jax.experimental.pallas import tpu as pltpu

_NC = 100
_B = 65536
_CH = 1024
_NBUF = 8
_G = _B // (_CH * _NBUF)


def _chunk_dma(x_hbm, xbuf, sems, c, k):
    return pltpu.make_async_copy(
        x_hbm.at[pl.ds(c * _CH, _CH), :], xbuf.at[k], sems.at[k])


def _body(x_hbm, t_ref, w_ref, out_ref, xbuf, acc_ref, sems):
    i = pl.program_id(0)

    @pl.when(i == 0)
    def _():
        acc_ref[...] = jnp.zeros_like(acc_ref)
        for b in range(_NBUF):
            _chunk_dma(x_hbm, xbuf, sems, b, b).start()

    w = w_ref[...]
    ones_row = jnp.ones((1, _NC), jnp.float32)
    cls = jax.lax.broadcasted_iota(jnp.int32, (_CH, _NC), 1)
    cls128 = jax.lax.broadcasted_iota(jnp.int32, (_CH, 128), 1)

    for k in range(_NBUF):
        c = i * _NBUF + k
        _chunk_dma(x_hbm, xbuf, sems, c, k).wait()
        x = xbuf[k]
        t = t_ref[pl.ds(k * _CH, _CH), :]
        e = jnp.exp(x) * w
        e_masked = jnp.where(t == cls, e, 0.0)
        s = jax.lax.dot_general(ones_row, e, (((1,), (1,)), ((), ())),
                                preferred_element_type=jnp.float32)
        et = jax.lax.dot_general(ones_row, e_masked, (((1,), (1,)), ((), ())),
                                 preferred_element_type=jnp.float32)
        p = jnp.clip(et / s, 1e-5, 1.0)

        oh128 = (t == cls128).astype(jnp.float32)
        pstack = jnp.concatenate([p, jnp.ones_like(p)], axis=0)
        part = jax.lax.dot_general(pstack, oh128, (((1,), (0,)), ((), ())),
                                   preferred_element_type=jnp.float32)
        acc_ref[...] += part

        @pl.when(i < _G - 1)
        def _():
            _chunk_dma(x_hbm, xbuf, sems, c + _NBUF, k).start()

    @pl.when(i == _G - 1)
    def _():
        sums = acc_ref[0:1, :]
        counts = acc_ref[1:2, :]
        exist = counts != 0.0
        denom = jnp.where(exist, counts, 1.0)
        meanp = sums / denom
        safe = jnp.where(exist, meanp, 1.0)
        ml = -jnp.log(safe)
        pw = jnp.where(exist, ml * ml * ml, 0.0)
        n_exist = jnp.sum(exist.astype(jnp.float32))
        msum = jnp.sum(pw) / n_exist
        loss = jnp.exp(jnp.log(msum) / 3.0)
        out_ref[...] = jnp.broadcast_to(loss, (1, 1))


def kernel(output, target, weight):
    res = pl.pallas_call(
        _body,
        grid=(_G,),
        in_specs=[
            pl.BlockSpec(memory_space=pl.ANY),
            pl.BlockSpec((_CH * _NBUF, 1), lambda i: (i, 0)),
            pl.BlockSpec((1, _NC), lambda i: (0, 0)),
        ],
        out_specs=pl.BlockSpec((1, 1), lambda i: (0, 0)),
        out_shape=jax.ShapeDtypeStruct((1, 1), jnp.float32),
        scratch_shapes=[
            pltpu.VMEM((_NBUF, _CH, _NC), jnp.float32),
            pltpu.VMEM((2, 128), jnp.float32),
            pltpu.SemaphoreType.DMA((_NBUF,)),
        ],
        compiler_params=pltpu.CompilerParams(
            dimension_semantics=("arbitrary",)),
    )(output, target.reshape(_B, 1), weight.reshape(1, _NC))
    return res[0, 0]
